# Initial kernel scaffold; baseline (speedup 1.0000x reference)
import jax
import jax.numpy as jnp
from jax.experimental import pallas as pl


def kernel(x, edge_index, W1, b1, W2, b2, W3, b3, W_out, b_out):
    raise NotImplementedError("write your pallas kernel here")


# speedup vs baseline: 5.6182x; 5.6182x over previous
import functools

import jax
import jax.numpy as jnp
from jax import lax
from jax.experimental import pallas as pl
from jax.experimental.pallas import tpu as pltpu
from jax.experimental.pallas import tpu_sc as plsc

N = 10000
E = 160000
NP = 10240
EP = 163840
CH = 128
NCHUNK = EP // CH
RB = 1000
GRID = N // RB

f32 = jnp.float32
i32 = jnp.int32

_mesh = plsc.VectorSubcoreMesh(core_axis_name="c", subcore_axis_name="s")


def _fill(ref, rows, width, value):
    val = jnp.full((16,), value, dtype=f32)

    @pl.loop(0, rows)
    def _(r):
        for j in range(width // 16):
            ref[r, pl.ds(16 * j, 16)] = val


@functools.partial(
    pl.kernel,
    out_type=jax.ShapeDtypeStruct((2, NP, 128), f32),
    mesh=_mesh,
    scratch_types=[
        pltpu.VMEM((CH, 128), f32),
        pltpu.VMEM((CH, 128), f32),
        pltpu.VMEM((CH,), i32),
        pltpu.VMEM((CH,), i32),
        pltpu.VMEM_SHARED((NP, 128), f32),
    ],
)
def _sc_degree(dst_hbm, out_hbm, zero_v, ones_v, idx0, idx1, slab):
    c = lax.axis_index("c")
    s = lax.axis_index("s")
    rows_per_tile = NP // 16
    row0 = s * rows_per_tile

    _fill(zero_v, CH, 128, 0.0)
    _fill(ones_v, CH, 128, 1.0)
    for k in range(rows_per_tile // CH):
        pltpu.sync_copy(zero_v, slab.at[pl.ds(row0 + k * CH, CH)])
    plsc.subcore_barrier()

    chunks_per_tile = NCHUNK // 32
    base = (c * 16 + s) * chunks_per_tile

    @pl.loop(0, chunks_per_tile // 2)
    def _(i):
        pltpu.sync_copy(dst_hbm.at[base + 2 * i], idx0)
        pltpu.sync_copy(ones_v, slab.at[idx0], add=True)
        pltpu.sync_copy(dst_hbm.at[base + 2 * i + 1], idx1)
        pltpu.sync_copy(ones_v, slab.at[idx1], add=True)

    plsc.subcore_barrier()

    for k in range(rows_per_tile // CH):
        r = row0 + k * CH
        pltpu.sync_copy(slab.at[pl.ds(r, CH)], zero_v)
        pltpu.sync_copy(zero_v, out_hbm.at[c].at[pl.ds(r, CH)])


def _make_prop(col_split):
    F = 128
    chunks_per_tile = NCHUNK // 16 if col_split else NCHUNK // 32
    rows_per_tile = NP // 16

    @functools.partial(
        pl.kernel,
        out_type=jax.ShapeDtypeStruct((2, NP, F), f32),
        mesh=_mesh,
        scratch_types=[
            pltpu.VMEM((CH, F), f32),
            pltpu.VMEM((CH, F), f32),
            pltpu.VMEM((CH,), i32),
            pltpu.VMEM((CH,), i32),
            pltpu.VMEM((CH,), i32),
            pltpu.VMEM((CH,), i32),
            pltpu.VMEM_SHARED((NP, F), f32),
            pltpu.SemaphoreType.DMA,
            pltpu.SemaphoreType.DMA,
        ],
    )
    def prop(g_hbm, src_hbm, dst_hbm, out_hbm, rows0, rows1, si0, si1,
             di0, di1, slab, sem0, sem1):
        c = lax.axis_index("c")
        s = lax.axis_index("s")
        row0 = s * rows_per_tile

        _fill(rows0, CH, F, 0.0)
        for k in range(rows_per_tile // CH):
            pltpu.sync_copy(rows0, slab.at[pl.ds(row0 + k * CH, CH)])
        plsc.subcore_barrier()

        if col_split:
            base = s * chunks_per_tile
            gsrc = g_hbm.at[c]
        else:
            base = (c * 16 + s) * chunks_per_tile
            gsrc = g_hbm
        rows = (rows0, rows1)
        sidx = (si0, si1)
        didx = (di0, di1)
        sems = (sem0, sem1)

        def start(i, b):
            pltpu.sync_copy(src_hbm.at[base + i], sidx[b])
            pltpu.sync_copy(dst_hbm.at[base + i], didx[b])
            return pltpu.async_copy(gsrc.at[sidx[b]], rows[b], sems[b])

        start(0, 0).wait()

        @pl.loop(0, chunks_per_tile - 1)
        def _(i):
            b = lax.rem(i, 2)

            @pl.when(b == 0)
            def _():
                d = start(i + 1, 1)
                pltpu.sync_copy(rows0, slab.at[di0], add=True)
                d.wait()

            @pl.when(b == 1)
            def _():
                d = start(i + 1, 0)
                pltpu.sync_copy(rows1, slab.at[di1], add=True)
                d.wait()

        last = (chunks_per_tile - 1) % 2
        pltpu.sync_copy(rows[last], slab.at[didx[last]], add=True)
        plsc.subcore_barrier()

        for k in range(rows_per_tile // CH):
            r = row0 + k * CH
            pltpu.sync_copy(slab.at[pl.ds(r, CH)], rows0)
            pltpu.sync_copy(rows0, out_hbm.at[c].at[pl.ds(r, CH)])

    return prop


_prop_cols = _make_prop(True)
_prop_edges = _make_prop(False)


def _dot(a, b):
    return jnp.dot(a, b, precision=lax.Precision.HIGHEST,
                   preferred_element_type=f32)


def _tc_first_body(p_ref, x_ref, w_ref, g_ref, dis_ref):
    deg = 1.0 + p_ref[0, :, 0:1] + p_ref[1, :, 0:1]
    dis = lax.rsqrt(deg)
    gt = _dot(x_ref[...], w_ref[...]) * dis
    g_ref[0] = gt[:, :128]
    g_ref[1] = gt[:, 128:]
    dis_ref[...] = dis


def _tc_first(p, x, w):
    return pl.pallas_call(
        _tc_first_body,
        grid=(GRID,),
        in_specs=[
            pl.BlockSpec((2, RB, 128), lambda i: (0, i, 0)),
            pl.BlockSpec((RB, 256), lambda i: (i, 0)),
            pl.BlockSpec((256, 256), lambda i: (0, 0)),
        ],
        out_specs=[
            pl.BlockSpec((2, RB, 128), lambda i: (0, i, 0)),
            pl.BlockSpec((RB, 1), lambda i: (i, 0)),
        ],
        out_shape=[
            jax.ShapeDtypeStruct((2, N, 128), f32),
            jax.ShapeDtypeStruct((N, 1), f32),
        ],
    )(p, x, w)


def _tc_mid_body(pad_out, s_ref, g_ref, dis_ref, b_ref, w_ref, o_ref):
    h = jnp.concatenate(
        [s_ref[0] + g_ref[0], s_ref[1] + g_ref[1]], axis=1)
    dis = dis_ref[...]
    h = jnp.maximum(h * dis + b_ref[...], 0.0)
    gt = _dot(h, w_ref[...]) * dis
    if pad_out:
        o_ref[...] = jnp.concatenate(
            [gt, jnp.zeros((gt.shape[0], 128 - gt.shape[1]), f32)], axis=1)
    else:
        half = w_ref.shape[1] // 2
        o_ref[0] = gt[:, :half]
        o_ref[1] = gt[:, half:]


def _tc_mid(s, g, dis, b, w):
    dout = w.shape[1]
    pad_out = dout < 256
    if pad_out:
        out_spec = pl.BlockSpec((RB, 128), lambda i: (i, 0))
        out_shape = jax.ShapeDtypeStruct((N, 128), f32)
    else:
        out_spec = pl.BlockSpec((2, RB, 128), lambda i: (0, i, 0))
        out_shape = jax.ShapeDtypeStruct((2, N, 128), f32)
    return pl.pallas_call(
        functools.partial(_tc_mid_body, pad_out),
        grid=(GRID,),
        in_specs=[
            pl.BlockSpec((2, RB, 128), lambda i: (0, i, 0)),
            pl.BlockSpec((2, RB, 128), lambda i: (0, i, 0)),
            pl.BlockSpec((RB, 1), lambda i: (i, 0)),
            pl.BlockSpec((1, 256), lambda i: (0, 0)),
            pl.BlockSpec((256, dout), lambda i: (0, 0)),
        ],
        out_specs=out_spec,
        out_shape=out_shape,
    )(s, g, dis, b, w)


def _tc_last_body(s_ref, g_ref, dis_ref, b_ref, o_ref):
    h = s_ref[0, :, :64] + s_ref[1, :, :64] + g_ref[:, :64]
    o_ref[...] = h * dis_ref[...] + b_ref[...]


def _tc_last(s, g, dis, b):
    return pl.pallas_call(
        _tc_last_body,
        grid=(GRID,),
        in_specs=[
            pl.BlockSpec((2, RB, 128), lambda i: (0, i, 0)),
            pl.BlockSpec((RB, 128), lambda i: (i, 0)),
            pl.BlockSpec((RB, 1), lambda i: (i, 0)),
            pl.BlockSpec((1, 64), lambda i: (0, 0)),
        ],
        out_specs=pl.BlockSpec((RB, 64), lambda i: (i, 0)),
        out_shape=jax.ShapeDtypeStruct((N, 64), f32),
    )(s, g, dis, b)


@jax.jit
def kernel(x, edge_index, W1, b1, W2, b2, W3, b3, W_out, b_out):
    src = edge_index[0].astype(i32)
    dst = edge_index[1].astype(i32)
    pad = EP - E
    src_p = jnp.concatenate([src, jnp.zeros((pad,), i32)])
    dst_p = jnp.concatenate(
        [dst, N + (jnp.arange(pad, dtype=i32) % (NP - N))])
    src2d = src_p.reshape(NCHUNK, CH)
    dst2d = dst_p.reshape(NCHUNK, CH)

    p = _sc_degree(dst2d)

    g1, dis = _tc_first(p, x, W1)
    s1 = _prop_cols(g1, src2d, dst2d)

    g2 = _tc_mid(s1, g1, dis, b1.reshape(1, -1), W2)
    s2 = _prop_cols(g2, src2d, dst2d)

    g3 = _tc_mid(s2, g2, dis, b2.reshape(1, -1), W3)
    s3 = _prop_cols(g3, src2d, dst2d)

    g4 = _tc_mid(s3, g3, dis, b3.reshape(1, -1), W_out)
    s4 = _prop_edges(g4, src2d, dst2d)

    return _tc_last(s4, g4, dis, b_out.reshape(1, -1))


# speedup vs baseline: 6.8808x; 1.2247x over previous
import functools

import jax
import jax.numpy as jnp
from jax import lax
from jax.experimental import pallas as pl
from jax.experimental.pallas import tpu as pltpu
from jax.experimental.pallas import tpu_sc as plsc

N = 10000
E = 160000
NP = 10240
EP = 163840
CH = 128
NCHUNK = EP // CH
RB = 1000
GRID = N // RB

f32 = jnp.float32
i32 = jnp.int32

_mesh = plsc.VectorSubcoreMesh(core_axis_name="c", subcore_axis_name="s")


def _fill(ref, rows, width, value):
    val = jnp.full((16,), value, dtype=f32)

    @pl.loop(0, rows)
    def _(r):
        for j in range(width // 16):
            ref[r, pl.ds(16 * j, 16)] = val


@functools.partial(
    pl.kernel,
    out_type=jax.ShapeDtypeStruct((2, NP, 128), f32),
    mesh=_mesh,
    scratch_types=[
        pltpu.VMEM((CH, 128), f32),
        pltpu.VMEM((CH, 128), f32),
        pltpu.VMEM((NCHUNK // 32, CH), i32),
        pltpu.VMEM_SHARED((NP, 128), f32),
    ],
)
def _sc_degree(dst_hbm, out_hbm, zero_v, ones_v, didx, slab):
    c = lax.axis_index("c")
    s = lax.axis_index("s")
    rows_per_tile = NP // 16
    row0 = s * rows_per_tile

    _fill(zero_v, CH, 128, 0.0)
    _fill(ones_v, CH, 128, 1.0)
    for k in range(rows_per_tile // CH):
        pltpu.sync_copy(zero_v, slab.at[pl.ds(row0 + k * CH, CH)])
    plsc.subcore_barrier()

    chunks_per_tile = NCHUNK // 32
    base = (c * 16 + s) * chunks_per_tile
    pltpu.sync_copy(dst_hbm.at[pl.ds(base, chunks_per_tile)], didx)

    @pl.loop(0, chunks_per_tile)
    def _(i):
        pltpu.sync_copy(ones_v, slab.at[didx.at[i]], add=True)

    plsc.subcore_barrier()

    for k in range(rows_per_tile // CH):
        r = row0 + k * CH
        pltpu.sync_copy(slab.at[pl.ds(r, CH)], zero_v)
        pltpu.sync_copy(zero_v, out_hbm.at[c].at[pl.ds(r, CH)])


def _make_prop(col_split):
    F = 128
    chunks_per_tile = NCHUNK // 16 if col_split else NCHUNK // 32
    nhalves = 2 if col_split else 1
    half = chunks_per_tile // nhalves
    rows_per_tile = NP // 16

    @functools.partial(
        pl.kernel,
        out_type=jax.ShapeDtypeStruct((2, NP, F), f32),
        mesh=_mesh,
        scratch_types=[
            pltpu.VMEM((CH, F), f32),
            pltpu.VMEM((CH, F), f32),
            pltpu.VMEM((half, CH), i32),
            pltpu.VMEM((half, CH), i32),
            pltpu.VMEM_SHARED((NP, F), f32),
            pltpu.SemaphoreType.DMA,
            pltpu.SemaphoreType.DMA,
        ],
    )
    def prop(g_hbm, src_hbm, dst_hbm, out_hbm, rows0, rows1, sidx, didx,
             slab, sem0, sem1):
        c = lax.axis_index("c")
        s = lax.axis_index("s")
        row0 = s * rows_per_tile
        rows = (rows0, rows1)
        sems = (sem0, sem1)

        _fill(rows0, CH, F, 0.0)
        for k in range(rows_per_tile // CH):
            pltpu.sync_copy(rows0, slab.at[pl.ds(row0 + k * CH, CH)])
        plsc.subcore_barrier()

        if col_split:
            base = s * chunks_per_tile
            gsrc = g_hbm.at[c]
        else:
            base = (c * 16 + s) * chunks_per_tile
            gsrc = g_hbm

        def gather(k, b):
            pltpu.async_copy(gsrc.at[sidx.at[k]], rows[b], sems[b])

        def wait_scat(k, b):
            pltpu.make_async_copy(gsrc.at[sidx.at[k]], rows[b], sems[b]).wait()
            pltpu.sync_copy(rows[b], slab.at[didx.at[k]], add=True)

        for h in range(nhalves):
            hb = base + h * half
            pltpu.sync_copy(src_hbm.at[pl.ds(hb, half)], sidx)
            pltpu.sync_copy(dst_hbm.at[pl.ds(hb, half)], didx)
            gather(0, 0)
            gather(1, 1)

            @pl.loop(0, half // 2 - 1)
            def _(i):
                k = 2 * i
                wait_scat(k, 0)
                gather(k + 2, 0)
                wait_scat(k + 1, 1)
                gather(k + 3, 1)

            wait_scat(half - 2, 0)
            wait_scat(half - 1, 1)

        plsc.subcore_barrier()

        for k in range(rows_per_tile // CH):
            r = row0 + k * CH
            pltpu.sync_copy(slab.at[pl.ds(r, CH)], rows[0])
            pltpu.sync_copy(rows[0], out_hbm.at[c].at[pl.ds(r, CH)])

    return prop


_prop_cols = _make_prop(True)
_prop_edges = _make_prop(False)


def _dot(a, b):
    return jnp.dot(a, b, precision=lax.Precision.HIGHEST,
                   preferred_element_type=f32)


def _tc_first_body(p_ref, x_ref, w_ref, g_ref, dis_ref):
    deg = 1.0 + p_ref[0, :, 0:1] + p_ref[1, :, 0:1]
    dis = lax.rsqrt(deg)
    gt = _dot(x_ref[...], w_ref[...]) * dis
    g_ref[0] = gt[:, :128]
    g_ref[1] = gt[:, 128:]
    dis_ref[...] = dis


def _tc_first(p, x, w):
    return pl.pallas_call(
        _tc_first_body,
        grid=(GRID,),
        in_specs=[
            pl.BlockSpec((2, RB, 128), lambda i: (0, i, 0)),
            pl.BlockSpec((RB, 256), lambda i: (i, 0)),
            pl.BlockSpec((256, 256), lambda i: (0, 0)),
        ],
        out_specs=[
            pl.BlockSpec((2, RB, 128), lambda i: (0, i, 0)),
            pl.BlockSpec((RB, 1), lambda i: (i, 0)),
        ],
        out_shape=[
            jax.ShapeDtypeStruct((2, N, 128), f32),
            jax.ShapeDtypeStruct((N, 1), f32),
        ],
    )(p, x, w)


def _tc_mid_body(pad_out, s_ref, g_ref, dis_ref, b_ref, w_ref, o_ref):
    h = jnp.concatenate(
        [s_ref[0] + g_ref[0], s_ref[1] + g_ref[1]], axis=1)
    dis = dis_ref[...]
    h = jnp.maximum(h * dis + b_ref[...], 0.0)
    gt = _dot(h, w_ref[...]) * dis
    if pad_out:
        o_ref[...] = jnp.concatenate(
            [gt, jnp.zeros((gt.shape[0], 128 - gt.shape[1]), f32)], axis=1)
    else:
        half = w_ref.shape[1] // 2
        o_ref[0] = gt[:, :half]
        o_ref[1] = gt[:, half:]


def _tc_mid(s, g, dis, b, w):
    dout = w.shape[1]
    pad_out = dout < 256
    if pad_out:
        out_spec = pl.BlockSpec((RB, 128), lambda i: (i, 0))
        out_shape = jax.ShapeDtypeStruct((N, 128), f32)
    else:
        out_spec = pl.BlockSpec((2, RB, 128), lambda i: (0, i, 0))
        out_shape = jax.ShapeDtypeStruct((2, N, 128), f32)
    return pl.pallas_call(
        functools.partial(_tc_mid_body, pad_out),
        grid=(GRID,),
        in_specs=[
            pl.BlockSpec((2, RB, 128), lambda i: (0, i, 0)),
            pl.BlockSpec((2, RB, 128), lambda i: (0, i, 0)),
            pl.BlockSpec((RB, 1), lambda i: (i, 0)),
            pl.BlockSpec((1, 256), lambda i: (0, 0)),
            pl.BlockSpec((256, dout), lambda i: (0, 0)),
        ],
        out_specs=out_spec,
        out_shape=out_shape,
    )(s, g, dis, b, w)


def _tc_last_body(s_ref, g_ref, dis_ref, b_ref, o_ref):
    h = s_ref[0, :, :64] + s_ref[1, :, :64] + g_ref[:, :64]
    o_ref[...] = h * dis_ref[...] + b_ref[...]


def _tc_last(s, g, dis, b):
    return pl.pallas_call(
        _tc_last_body,
        grid=(GRID,),
        in_specs=[
            pl.BlockSpec((2, RB, 128), lambda i: (0, i, 0)),
            pl.BlockSpec((RB, 128), lambda i: (i, 0)),
            pl.BlockSpec((RB, 1), lambda i: (i, 0)),
            pl.BlockSpec((1, 64), lambda i: (0, 0)),
        ],
        out_specs=pl.BlockSpec((RB, 64), lambda i: (i, 0)),
        out_shape=jax.ShapeDtypeStruct((N, 64), f32),
    )(s, g, dis, b)


@jax.jit
def kernel(x, edge_index, W1, b1, W2, b2, W3, b3, W_out, b_out):
    src = edge_index[0].astype(i32)
    dst = edge_index[1].astype(i32)
    pad = EP - E
    src_p = jnp.concatenate([src, jnp.zeros((pad,), i32)])
    dst_p = jnp.concatenate(
        [dst, N + (jnp.arange(pad, dtype=i32) % (NP - N))])
    src2d = src_p.reshape(NCHUNK, CH)
    dst2d = dst_p.reshape(NCHUNK, CH)

    p = _sc_degree(dst2d)

    g1, dis = _tc_first(p, x, W1)
    s1 = _prop_cols(g1, src2d, dst2d)

    g2 = _tc_mid(s1, g1, dis, b1.reshape(1, -1), W2)
    s2 = _prop_cols(g2, src2d, dst2d)

    g3 = _tc_mid(s2, g2, dis, b2.reshape(1, -1), W3)
    s3 = _prop_cols(g3, src2d, dst2d)

    g4 = _tc_mid(s3, g3, dis, b3.reshape(1, -1), W_out)
    s4 = _prop_edges(g4, src2d, dst2d)

    return _tc_last(s4, g4, dis, b_out.reshape(1, -1))


# speedup vs baseline: 16.3379x; 2.3744x over previous
import functools

import jax
import jax.numpy as jnp
from jax import lax
from jax.experimental import pallas as pl
from jax.experimental.pallas import tpu as pltpu
from jax.experimental.pallas import tpu_sc as plsc

N = 10000
E = 160000
NP = 10240
EP = 163840
CH = 128
NCHUNK = EP // CH
RB = 1000
GRID = N // RB

f32 = jnp.float32
i32 = jnp.int32

_mesh = plsc.VectorSubcoreMesh(core_axis_name="c", subcore_axis_name="s")


def _fill(ref, rows, width, value):
    val = jnp.full((16,), value, dtype=f32)

    @pl.loop(0, rows)
    def _(r):
        for j in range(width // 16):
            ref[r, pl.ds(16 * j, 16)] = val


@functools.partial(
    pl.kernel,
    out_type=jax.ShapeDtypeStruct((2, NP, 128), f32),
    mesh=_mesh,
    scratch_types=[
        pltpu.VMEM((CH, 128), f32),
        pltpu.VMEM((CH, 128), f32),
        pltpu.VMEM((NCHUNK // 32, CH), i32),
        pltpu.VMEM_SHARED((NP, 128), f32),
    ],
)
def _sc_degree(dst_hbm, out_hbm, zero_v, ones_v, didx, slab):
    c = lax.axis_index("c")
    s = lax.axis_index("s")
    rows_per_tile = NP // 16
    row0 = s * rows_per_tile

    _fill(zero_v, CH, 128, 0.0)
    _fill(ones_v, CH, 128, 1.0)
    for k in range(rows_per_tile // CH):
        pltpu.sync_copy(zero_v, slab.at[pl.ds(row0 + k * CH, CH)])
    plsc.subcore_barrier()

    chunks_per_tile = NCHUNK // 32
    base = (c * 16 + s) * chunks_per_tile
    pltpu.sync_copy(dst_hbm.at[pl.ds(base, chunks_per_tile)], didx)

    @pl.loop(0, chunks_per_tile)
    def _(i):
        pltpu.sync_copy(ones_v, slab.at[didx.at[i]], add=True)

    plsc.subcore_barrier()

    pltpu.sync_copy(slab.at[pl.ds(row0, rows_per_tile)],
                    out_hbm.at[c].at[pl.ds(row0, rows_per_tile)])


def _make_prop(col_split):
    F = 128
    chunks_per_tile = NCHUNK // 16 if col_split else NCHUNK // 32
    nhalves = 2 if col_split else 1
    half = chunks_per_tile // nhalves
    rows_per_tile = NP // 16

    @functools.partial(
        pl.kernel,
        out_type=jax.ShapeDtypeStruct((2, NP, F), f32),
        mesh=_mesh,
        scratch_types=[
            pltpu.VMEM((CH, F), f32),
            pltpu.VMEM((CH, F), f32),
            pltpu.VMEM((half, CH), i32),
            pltpu.VMEM((half, CH), i32),
            pltpu.VMEM_SHARED((NP, F), f32),
            pltpu.SemaphoreType.DMA,
            pltpu.SemaphoreType.DMA,
        ],
    )
    def prop(g_hbm, src_hbm, dst_hbm, out_hbm, rows0, rows1, sidx, didx,
             slab, sem0, sem1):
        c = lax.axis_index("c")
        s = lax.axis_index("s")
        row0 = s * rows_per_tile
        rows = (rows0, rows1)
        sems = (sem0, sem1)

        _fill(rows0, CH, F, 0.0)
        for k in range(rows_per_tile // CH):
            pltpu.sync_copy(rows0, slab.at[pl.ds(row0 + k * CH, CH)])
        plsc.subcore_barrier()

        if col_split:
            base = s * chunks_per_tile
            gsrc = g_hbm.at[c]
        else:
            base = (c * 16 + s) * chunks_per_tile
            gsrc = g_hbm

        def gather(k, b):
            pltpu.async_copy(gsrc.at[sidx.at[k]], rows[b], sems[b])

        def wait_scat(k, b):
            pltpu.make_async_copy(gsrc.at[sidx.at[k]], rows[b], sems[b]).wait()
            pltpu.sync_copy(rows[b], slab.at[didx.at[k]], add=True)

        for h in range(nhalves):
            hb = base + h * half
            pltpu.sync_copy(src_hbm.at[pl.ds(hb, half)], sidx)
            pltpu.sync_copy(dst_hbm.at[pl.ds(hb, half)], didx)
            gather(0, 0)
            gather(1, 1)

            @pl.loop(0, half // 2 - 1)
            def _(i):
                k = 2 * i
                wait_scat(k, 0)
                gather(k + 2, 0)
                wait_scat(k + 1, 1)
                gather(k + 3, 1)

            wait_scat(half - 2, 0)
            wait_scat(half - 1, 1)

        plsc.subcore_barrier()

        pltpu.sync_copy(slab.at[pl.ds(row0, rows_per_tile)],
                        out_hbm.at[c].at[pl.ds(row0, rows_per_tile)])

    return prop


_prop_cols = _make_prop(True)
_prop_edges = _make_prop(False)


def _dot(a, b):
    return jnp.dot(a, b, precision=lax.Precision.HIGHEST,
                   preferred_element_type=f32)


def _tc_first_body(p_ref, x_ref, w_ref, g_ref, dis_ref):
    deg = 1.0 + p_ref[0, :, 0:1] + p_ref[1, :, 0:1]
    dis = lax.rsqrt(deg)
    gt = _dot(x_ref[...], w_ref[...]) * dis
    g_ref[0] = gt[:, :128]
    g_ref[1] = gt[:, 128:]
    dis_ref[...] = dis


def _tc_first(p, x, w):
    return pl.pallas_call(
        _tc_first_body,
        grid=(GRID,),
        in_specs=[
            pl.BlockSpec((2, RB, 128), lambda i: (0, i, 0)),
            pl.BlockSpec((RB, 256), lambda i: (i, 0)),
            pl.BlockSpec((256, 256), lambda i: (0, 0)),
        ],
        out_specs=[
            pl.BlockSpec((2, RB, 128), lambda i: (0, i, 0)),
            pl.BlockSpec((RB, 1), lambda i: (i, 0)),
        ],
        out_shape=[
            jax.ShapeDtypeStruct((2, N, 128), f32),
            jax.ShapeDtypeStruct((N, 1), f32),
        ],
    )(p, x, w)


def _tc_mid_body(pad_out, s_ref, g_ref, dis_ref, b_ref, w_ref, o_ref):
    h = jnp.concatenate(
        [s_ref[0] + g_ref[0], s_ref[1] + g_ref[1]], axis=1)
    dis = dis_ref[...]
    h = jnp.maximum(h * dis + b_ref[...], 0.0)
    gt = _dot(h, w_ref[...]) * dis
    if pad_out:
        o_ref[...] = jnp.concatenate(
            [gt, jnp.zeros((gt.shape[0], 128 - gt.shape[1]), f32)], axis=1)
    else:
        half = w_ref.shape[1] // 2
        o_ref[0] = gt[:, :half]
        o_ref[1] = gt[:, half:]


def _tc_mid(s, g, dis, b, w):
    dout = w.shape[1]
    pad_out = dout < 256
    if pad_out:
        out_spec = pl.BlockSpec((RB, 128), lambda i: (i, 0))
        out_shape = jax.ShapeDtypeStruct((N, 128), f32)
    else:
        out_spec = pl.BlockSpec((2, RB, 128), lambda i: (0, i, 0))
        out_shape = jax.ShapeDtypeStruct((2, N, 128), f32)
    return pl.pallas_call(
        functools.partial(_tc_mid_body, pad_out),
        grid=(GRID,),
        in_specs=[
            pl.BlockSpec((2, RB, 128), lambda i: (0, i, 0)),
            pl.BlockSpec((2, RB, 128), lambda i: (0, i, 0)),
            pl.BlockSpec((RB, 1), lambda i: (i, 0)),
            pl.BlockSpec((1, 256), lambda i: (0, 0)),
            pl.BlockSpec((256, dout), lambda i: (0, 0)),
        ],
        out_specs=out_spec,
        out_shape=out_shape,
    )(s, g, dis, b, w)


def _tc_last_body(s_ref, g_ref, dis_ref, b_ref, o_ref):
    h = s_ref[0, :, :64] + s_ref[1, :, :64] + g_ref[:, :64]
    o_ref[...] = h * dis_ref[...] + b_ref[...]


def _tc_last(s, g, dis, b):
    return pl.pallas_call(
        _tc_last_body,
        grid=(GRID,),
        in_specs=[
            pl.BlockSpec((2, RB, 128), lambda i: (0, i, 0)),
            pl.BlockSpec((RB, 128), lambda i: (i, 0)),
            pl.BlockSpec((RB, 1), lambda i: (i, 0)),
            pl.BlockSpec((1, 64), lambda i: (0, 0)),
        ],
        out_specs=pl.BlockSpec((RB, 64), lambda i: (i, 0)),
        out_shape=jax.ShapeDtypeStruct((N, 64), f32),
    )(s, g, dis, b)


@jax.jit
def kernel(x, edge_index, W1, b1, W2, b2, W3, b3, W_out, b_out):
    src = edge_index[0].astype(i32)
    dst = edge_index[1].astype(i32)
    npad = NCHUNK * (CH - E // NCHUNK)
    pad_src = (jnp.arange(npad, dtype=i32) * 7919) % N
    pad_dst = N + (jnp.arange(npad, dtype=i32) % (NP - N))
    src2d = jnp.concatenate(
        [src.reshape(NCHUNK, E // NCHUNK), pad_src.reshape(NCHUNK, -1)], axis=1)
    dst2d = jnp.concatenate(
        [dst.reshape(NCHUNK, E // NCHUNK), pad_dst.reshape(NCHUNK, -1)], axis=1)

    p = _sc_degree(dst2d)

    g1, dis = _tc_first(p, x, W1)
    s1 = _prop_cols(g1, src2d, dst2d)

    g2 = _tc_mid(s1, g1, dis, b1.reshape(1, -1), W2)
    s2 = _prop_cols(g2, src2d, dst2d)

    g3 = _tc_mid(s2, g2, dis, b2.reshape(1, -1), W3)
    s3 = _prop_cols(g3, src2d, dst2d)

    g4 = _tc_mid(s3, g3, dis, b3.reshape(1, -1), W_out)
    s4 = _prop_edges(g4, src2d, dst2d)

    return _tc_last(s4, g4, dis, b_out.reshape(1, -1))


# speedup vs baseline: 16.4188x; 1.0050x over previous
import functools

import jax
import jax.numpy as jnp
from jax import lax
from jax.experimental import pallas as pl
from jax.experimental.pallas import tpu as pltpu
from jax.experimental.pallas import tpu_sc as plsc

N = 10000
E = 160000
NP = 10240
EP = 163840
CH = 128
NCHUNK = EP // CH
RB = 1000
GRID = N // RB

f32 = jnp.float32
i32 = jnp.int32

_mesh = plsc.VectorSubcoreMesh(core_axis_name="c", subcore_axis_name="s")


def _fill(ref, rows, width, value):
    val = jnp.full((16,), value, dtype=f32)

    @pl.loop(0, rows)
    def _(r):
        for j in range(width // 16):
            ref[r, pl.ds(16 * j, 16)] = val


@functools.partial(
    pl.kernel,
    out_type=jax.ShapeDtypeStruct((2, NP, 128), f32),
    mesh=_mesh,
    scratch_types=[
        pltpu.VMEM((CH, 128), f32),
        pltpu.VMEM((CH, 128), f32),
        pltpu.VMEM((NCHUNK // 32, CH), i32),
        pltpu.VMEM_SHARED((NP, 128), f32),
        pltpu.SemaphoreType.DMA,
    ],
)
def _sc_degree(dst_hbm, out_hbm, zero_v, ones_v, didx, slab, sem):
    c = lax.axis_index("c")
    s = lax.axis_index("s")
    rows_per_tile = NP // 16
    row0 = s * rows_per_tile

    _fill(zero_v, CH, 128, 0.0)
    _fill(ones_v, CH, 128, 1.0)
    for k in range(rows_per_tile // CH):
        pltpu.async_copy(zero_v, slab.at[pl.ds(row0 + k * CH, CH)], sem)
    chunks_per_tile = NCHUNK // 32
    base = (c * 16 + s) * chunks_per_tile
    pltpu.sync_copy(dst_hbm.at[pl.ds(base, chunks_per_tile)], didx)
    for k in range(rows_per_tile // CH):
        pltpu.make_async_copy(zero_v,
                              slab.at[pl.ds(row0 + k * CH, CH)], sem).wait()
    plsc.subcore_barrier()

    @pl.loop(0, chunks_per_tile)
    def _(i):
        pltpu.async_copy(ones_v, slab.at[didx.at[i]], sem, add=True)

    @pl.loop(0, chunks_per_tile)
    def _(i):
        pltpu.make_async_copy(ones_v, slab.at[didx.at[i]], sem).wait()

    plsc.subcore_barrier()

    pltpu.sync_copy(slab.at[pl.ds(row0, rows_per_tile)],
                    out_hbm.at[c].at[pl.ds(row0, rows_per_tile)])


def _make_prop(col_split):
    F = 128
    chunks_per_tile = NCHUNK // 16 if col_split else NCHUNK // 32
    nhalves = 2 if col_split else 1
    half = chunks_per_tile // nhalves
    rows_per_tile = NP // 16

    @functools.partial(
        pl.kernel,
        out_type=jax.ShapeDtypeStruct((2, NP, F), f32),
        mesh=_mesh,
        scratch_types=[
            pltpu.VMEM((CH, F), f32),
            pltpu.VMEM((CH, F), f32),
            pltpu.VMEM((half, CH), i32),
            pltpu.VMEM((half, CH), i32),
            pltpu.VMEM_SHARED((NP, F), f32),
            pltpu.SemaphoreType.DMA,
            pltpu.SemaphoreType.DMA,
        ],
    )
    def prop(g_hbm, src_hbm, dst_hbm, out_hbm, rows0, rows1, sidx, didx,
             slab, sem0, sem1):
        c = lax.axis_index("c")
        s = lax.axis_index("s")
        row0 = s * rows_per_tile
        rows = (rows0, rows1)
        sems = (sem0, sem1)

        _fill(rows0, CH, F, 0.0)
        for k in range(rows_per_tile // CH):
            pltpu.async_copy(rows0, slab.at[pl.ds(row0 + k * CH, CH)], sem0)
        for k in range(rows_per_tile // CH):
            pltpu.make_async_copy(rows0,
                                  slab.at[pl.ds(row0 + k * CH, CH)],
                                  sem0).wait()
        plsc.subcore_barrier()

        if col_split:
            base = s * chunks_per_tile
            gsrc = g_hbm.at[c]
        else:
            base = (c * 16 + s) * chunks_per_tile
            gsrc = g_hbm

        def gather(k, b):
            pltpu.async_copy(gsrc.at[sidx.at[k]], rows[b], sems[b])

        def wait_scat(k, b):
            pltpu.make_async_copy(gsrc.at[sidx.at[k]], rows[b], sems[b]).wait()
            pltpu.sync_copy(rows[b], slab.at[didx.at[k]], add=True)

        for h in range(nhalves):
            hb = base + h * half
            pltpu.sync_copy(src_hbm.at[pl.ds(hb, half)], sidx)
            pltpu.sync_copy(dst_hbm.at[pl.ds(hb, half)], didx)
            gather(0, 0)
            gather(1, 1)

            @pl.loop(0, half // 2 - 1)
            def _(i):
                k = 2 * i
                wait_scat(k, 0)
                gather(k + 2, 0)
                wait_scat(k + 1, 1)
                gather(k + 3, 1)

            wait_scat(half - 2, 0)
            wait_scat(half - 1, 1)

        plsc.subcore_barrier()

        pltpu.sync_copy(slab.at[pl.ds(row0, rows_per_tile)],
                        out_hbm.at[c].at[pl.ds(row0, rows_per_tile)])

    return prop


_prop_cols = _make_prop(True)
_prop_edges = _make_prop(False)


def _dot(a, b):
    return jnp.dot(a, b, precision=lax.Precision.HIGHEST,
                   preferred_element_type=f32)


def _tc_first_body(p_ref, x_ref, w_ref, g_ref, dis_ref):
    deg = 1.0 + p_ref[0, :, 0:1] + p_ref[1, :, 0:1]
    dis = lax.rsqrt(deg)
    gt = _dot(x_ref[...], w_ref[...]) * dis
    g_ref[0] = gt[:, :128]
    g_ref[1] = gt[:, 128:]
    dis_ref[...] = dis


def _tc_first(p, x, w):
    return pl.pallas_call(
        _tc_first_body,
        grid=(GRID,),
        in_specs=[
            pl.BlockSpec((2, RB, 128), lambda i: (0, i, 0)),
            pl.BlockSpec((RB, 256), lambda i: (i, 0)),
            pl.BlockSpec((256, 256), lambda i: (0, 0)),
        ],
        out_specs=[
            pl.BlockSpec((2, RB, 128), lambda i: (0, i, 0)),
            pl.BlockSpec((RB, 1), lambda i: (i, 0)),
        ],
        out_shape=[
            jax.ShapeDtypeStruct((2, N, 128), f32),
            jax.ShapeDtypeStruct((N, 1), f32),
        ],
    )(p, x, w)


def _tc_mid_body(pad_out, s_ref, g_ref, dis_ref, b_ref, w_ref, o_ref):
    h = jnp.concatenate(
        [s_ref[0] + g_ref[0], s_ref[1] + g_ref[1]], axis=1)
    dis = dis_ref[...]
    h = jnp.maximum(h * dis + b_ref[...], 0.0)
    gt = _dot(h, w_ref[...]) * dis
    if pad_out:
        o_ref[...] = jnp.concatenate(
            [gt, jnp.zeros((gt.shape[0], 128 - gt.shape[1]), f32)], axis=1)
    else:
        half = w_ref.shape[1] // 2
        o_ref[0] = gt[:, :half]
        o_ref[1] = gt[:, half:]


def _tc_mid(s, g, dis, b, w):
    dout = w.shape[1]
    pad_out = dout < 256
    if pad_out:
        out_spec = pl.BlockSpec((RB, 128), lambda i: (i, 0))
        out_shape = jax.ShapeDtypeStruct((N, 128), f32)
    else:
        out_spec = pl.BlockSpec((2, RB, 128), lambda i: (0, i, 0))
        out_shape = jax.ShapeDtypeStruct((2, N, 128), f32)
    return pl.pallas_call(
        functools.partial(_tc_mid_body, pad_out),
        grid=(GRID,),
        in_specs=[
            pl.BlockSpec((2, RB, 128), lambda i: (0, i, 0)),
            pl.BlockSpec((2, RB, 128), lambda i: (0, i, 0)),
            pl.BlockSpec((RB, 1), lambda i: (i, 0)),
            pl.BlockSpec((1, 256), lambda i: (0, 0)),
            pl.BlockSpec((256, dout), lambda i: (0, 0)),
        ],
        out_specs=out_spec,
        out_shape=out_shape,
    )(s, g, dis, b, w)


def _tc_last_body(s_ref, g_ref, dis_ref, b_ref, o_ref):
    h = s_ref[0, :, :64] + s_ref[1, :, :64] + g_ref[:, :64]
    o_ref[...] = h * dis_ref[...] + b_ref[...]


def _tc_last(s, g, dis, b):
    return pl.pallas_call(
        _tc_last_body,
        grid=(GRID,),
        in_specs=[
            pl.BlockSpec((2, RB, 128), lambda i: (0, i, 0)),
            pl.BlockSpec((RB, 128), lambda i: (i, 0)),
            pl.BlockSpec((RB, 1), lambda i: (i, 0)),
            pl.BlockSpec((1, 64), lambda i: (0, 0)),
        ],
        out_specs=pl.BlockSpec((RB, 64), lambda i: (i, 0)),
        out_shape=jax.ShapeDtypeStruct((N, 64), f32),
    )(s, g, dis, b)


@jax.jit
def kernel(x, edge_index, W1, b1, W2, b2, W3, b3, W_out, b_out):
    src = edge_index[0].astype(i32)
    dst = edge_index[1].astype(i32)
    npad = NCHUNK * (CH - E // NCHUNK)
    pad_src = (jnp.arange(npad, dtype=i32) * 7919) % N
    pad_dst = N + (jnp.arange(npad, dtype=i32) % (NP - N))
    src2d = jnp.concatenate(
        [src.reshape(NCHUNK, E // NCHUNK), pad_src.reshape(NCHUNK, -1)], axis=1)
    dst2d = jnp.concatenate(
        [dst.reshape(NCHUNK, E // NCHUNK), pad_dst.reshape(NCHUNK, -1)], axis=1)

    p = _sc_degree(dst2d)

    g1, dis = _tc_first(p, x, W1)
    s1 = _prop_cols(g1, src2d, dst2d)

    g2 = _tc_mid(s1, g1, dis, b1.reshape(1, -1), W2)
    s2 = _prop_cols(g2, src2d, dst2d)

    g3 = _tc_mid(s2, g2, dis, b2.reshape(1, -1), W3)
    s3 = _prop_cols(g3, src2d, dst2d)

    g4 = _tc_mid(s3, g3, dis, b3.reshape(1, -1), W_out)
    s4 = _prop_edges(g4, src2d, dst2d)

    return _tc_last(s4, g4, dis, b_out.reshape(1, -1))
